# baseline (speedup 1.0000x reference)
import functools

import jax
import jax.numpy as jnp
from jax import lax
from jax.experimental import pallas as pl
from jax.experimental.pallas import tpu as pltpu
from jax.experimental.pallas import tpu_sc as plsc

NUM_ITEMS = 1000000
EMB = 64
BATCH = 4096
HIST = 200
NW = 32
ROWS_W = BATCH // NW


def _body(idx_hbm, tab_hbm, out_hbm, idxT, gv0, gv1, tv0, tv1, sI, sg0, sg1,
          ss0, ss1):
    cid = lax.axis_index("c")
    sid = lax.axis_index("s")
    wid = sid * 2 + cid
    lanes = lax.iota(jnp.int32, 16)

    pltpu.async_copy(idx_hbm.at[wid], idxT, sI)
    pltpu.make_async_copy(idx_hbm.at[wid], idxT, sI).wait()

    gv = (gv0, gv1)
    tv = (tv0, tv1)
    sg = (sg0, sg1)
    ss = (ss0, ss1)

    def start_gather(h, b):
        pltpu.async_copy(tab_hbm.at[idxT.at[h]], gv[b], sg[b])

    def wait_gather(h, b):
        pltpu.make_async_copy(tab_hbm.at[idxT.at[h]], gv[b], sg[b]).wait()

    tf_vecs = [(f0 + lanes) // 8 for f0 in range(0, EMB, 16)]
    fi_vecs = [(f0 + lanes) % 8 for f0 in range(0, EMB, 16)]
    zero16 = jnp.full((16,), 0, jnp.int32)

    def transpose_chunk(b):
        @pl.loop(0, ROWS_W // 4)
        def _(q):
            for u in range(4):
                bi = q * 4 + u
                bvec = zero16 + bi
                for i4 in range(4):
                    vec = gv[b][bi, pl.ds(i4 * 16, 16)]
                    plsc.store_scatter(
                        tv[b], [tf_vecs[i4], fi_vecs[i4], bvec], vec)

    def start_store(h, b):
        pltpu.async_copy(tv[b], out_hbm.at[h, :, wid], ss[b])

    def wait_store(h, b):
        pltpu.make_async_copy(tv[b], out_hbm.at[h, :, wid], ss[b]).wait()

    start_gather(0, 0)
    wait_gather(0, 0)
    start_gather(1, 1)
    transpose_chunk(0)
    start_store(0, 0)

    @pl.loop(0, (HIST - 2) // 2)
    def _(jj):
        h = 1 + 2 * jj
        wait_gather(h, 1)
        start_gather(h + 1, 0)
        transpose_chunk(1)
        wait_store(h - 1, 0)
        start_store(h, 1)
        wait_gather(h + 1, 0)

        @pl.when(h + 2 < HIST)
        def _():
            start_gather(h + 2, 1)
        transpose_chunk(0)
        wait_store(h, 1)
        start_store(h + 1, 0)

    wait_gather(HIST - 1, 1)
    transpose_chunk(1)
    wait_store(HIST - 2, 0)
    start_store(HIST - 1, 1)
    wait_store(HIST - 1, 1)


@jax.jit
def _emb_lookup(idxT3, tab128):
    mesh = plsc.VectorSubcoreMesh(core_axis_name="c", subcore_axis_name="s")
    f = functools.partial(
        pl.kernel,
        out_type=jax.ShapeDtypeStruct((HIST, 8, NW, 8, ROWS_W), jnp.float32),
        mesh=mesh,
        compiler_params=pltpu.CompilerParams(
            use_tc_tiling_on_sc=True, needs_layout_passes=False),
        scratch_types=[
            pltpu.VMEM((HIST, ROWS_W), jnp.int32),
            pltpu.VMEM((ROWS_W, 128), jnp.float32),
            pltpu.VMEM((ROWS_W, 128), jnp.float32),
            pltpu.VMEM((8, 8, ROWS_W), jnp.float32),
            pltpu.VMEM((8, 8, ROWS_W), jnp.float32),
            pltpu.SemaphoreType.DMA,
            pltpu.SemaphoreType.DMA,
            pltpu.SemaphoreType.DMA,
            pltpu.SemaphoreType.DMA,
            pltpu.SemaphoreType.DMA,
        ],
    )(_body)
    return f(idxT3, tab128)


def kernel(input_seqs, item_emb):
    tab128 = jnp.pad(item_emb, ((0, 0), (0, 128 - EMB)))
    idxT3 = input_seqs.reshape(NW, ROWS_W, HIST).transpose(0, 2, 1)
    out5d = _emb_lookup(idxT3, tab128)
    return jnp.transpose(out5d, (2, 4, 0, 1, 3)).reshape(BATCH, HIST, EMB)


# speedup vs baseline: 1.5103x; 1.5103x over previous
import functools

import jax
import jax.numpy as jnp
from jax import lax
from jax.experimental import pallas as pl
from jax.experimental.pallas import tpu as pltpu
from jax.experimental.pallas import tpu_sc as plsc

NUM_ITEMS = 1000000
EMB = 64
BATCH = 4096
HIST = 200
NW = 32
ROWS_W = BATCH // NW
PER_W = ROWS_W * HIST
NBUF = 4


def _emb_body(idx_hbm, tab_hbm, out_hbm, idx_v, rows_v,
              sg0, sg1, sg2, sg3, ss0, ss1, ss2, ss3):
    wid = lax.axis_index("s") * 2 + lax.axis_index("c")
    base = wid * ROWS_W

    pltpu.sync_copy(idx_hbm.at[wid], idx_v)

    sg = (sg0, sg1, sg2, sg3)
    ss = (ss0, ss1, ss2, ss3)

    def start_gather(i, b):
        pltpu.async_copy(
            tab_hbm.at[idx_v.at[pl.ds(i * HIST, HIST)]], rows_v.at[b], sg[b])

    def wait_gather(i, b):
        pltpu.make_async_copy(
            tab_hbm.at[idx_v.at[pl.ds(i * HIST, HIST)]], rows_v.at[b],
            sg[b]).wait()

    def start_store(i, b):
        pltpu.async_copy(rows_v.at[b], out_hbm.at[base + i], ss[b])

    def wait_store(i, b):
        pltpu.make_async_copy(rows_v.at[b], out_hbm.at[base + i],
                              ss[b]).wait()

    for k in range(NBUF - 1):
        start_gather(k, k)

    @pl.loop(0, ROWS_W // NBUF)
    def _(p):
        for k in range(NBUF):
            i = p * NBUF + k

            @pl.when(i >= 1)
            def _():
                wait_store(i - 1, (k - 1) % NBUF)

            @pl.when(i + NBUF - 1 < ROWS_W)
            def _():
                start_gather(i + NBUF - 1, (k + NBUF - 1) % NBUF)
            wait_gather(i, k)
            start_store(i, k)

    wait_store(ROWS_W - 1, (ROWS_W - 1) % NBUF)


@jax.jit
def _emb_lookup(idx32, tab128):
    mesh = plsc.VectorSubcoreMesh(core_axis_name="c", subcore_axis_name="s")
    f = functools.partial(
        pl.kernel,
        out_type=jax.ShapeDtypeStruct((BATCH, HIST, 128), jnp.float32),
        mesh=mesh,
        compiler_params=pltpu.CompilerParams(use_tc_tiling_on_sc=True),
        scratch_types=[
            pltpu.VMEM((PER_W,), jnp.int32),
            pltpu.VMEM((NBUF, HIST, 128), jnp.float32),
            pltpu.SemaphoreType.DMA,
            pltpu.SemaphoreType.DMA,
            pltpu.SemaphoreType.DMA,
            pltpu.SemaphoreType.DMA,
            pltpu.SemaphoreType.DMA,
            pltpu.SemaphoreType.DMA,
            pltpu.SemaphoreType.DMA,
            pltpu.SemaphoreType.DMA,
        ],
    )(_emb_body)
    return f(idx32, tab128)


def kernel(input_seqs, item_emb):
    tab128 = jnp.pad(item_emb, ((0, 0), (0, 128 - EMB)))
    idx32 = input_seqs.reshape(NW, PER_W)
    return _emb_lookup(idx32, tab128)[..., :EMB]
